# baseline (speedup 1.0000x reference)
import functools

import jax
import jax.numpy as jnp
from jax import lax
from jax.experimental import pallas as pl
from jax.experimental.pallas import tpu as pltpu
from jax.experimental.pallas import tpu_sc as plsc

B, N, D = 1024, 77, 768
NUM_NAMES = 1000
PLACEHOLDER_TOKEN = 265

_NC, _NS = 2, 16
_NW = _NC * _NS
_B_PER_W = B // _NW


def _sc_gather_body(name_hbm, params_hbm, out_hbm, idx_v, rows_v, sem):
    wid = lax.axis_index("s") * _NC + lax.axis_index("c")
    base = wid * _B_PER_W
    pltpu.sync_copy(name_hbm.at[pl.ds(base, _B_PER_W)], idx_v)
    pltpu.async_copy(params_hbm.at[idx_v], rows_v, sem).wait()
    pltpu.sync_copy(rows_v, out_hbm.at[pl.ds(base, _B_PER_W)])


@functools.cache
def _sc_gather():
    return pl.kernel(
        _sc_gather_body,
        out_type=jax.ShapeDtypeStruct((B, D), jnp.float32),
        mesh=plsc.VectorSubcoreMesh(core_axis_name="c", subcore_axis_name="s"),
        scratch_types=[
            pltpu.VMEM((_B_PER_W,), jnp.int32),
            pltpu.VMEM((_B_PER_W, D), jnp.float32),
            pltpu.SemaphoreType.DMA,
        ],
    )


BB = 16
NBLK = B // BB
NBUF = 4


def _in_copy(i, slot, emb_ref, buf, isem):
    return pltpu.make_async_copy(
        emb_ref.at[pl.ds(i * BB, BB)], buf.at[slot], isem.at[slot]
    )


def _out_copy(i, slot, out_ref, buf, osem):
    return pltpu.make_async_copy(
        buf.at[slot], out_ref.at[pl.ds(i * BB, BB)], osem.at[slot]
    )


def _pipe_body(tok_ref, emb_ref, g_ref, out_ref, buf, isem, osem):
    def step(i, carry):
        slot = lax.rem(i, NBUF)

        @pl.when(i < NBLK)
        def _():
            @pl.when(i >= NBUF)
            def _():
                _out_copy(i - NBUF, slot, out_ref, buf, osem).wait()

            _in_copy(i, slot, emb_ref, buf, isem).start()

        @pl.when(i >= 1)
        def _():
            j = i - 1
            jslot = lax.rem(j, NBUF)
            _in_copy(j, jslot, emb_ref, buf, isem).wait()
            base = j * BB
            tokblk = tok_ref[pl.ds(base, BB), :]
            for r in range(BB):
                m = tokblk[r : r + 1, :] == PLACEHOLDER_TOKEN
                col = jnp.sum(
                    jnp.where(m, lax.broadcasted_iota(jnp.int32, (1, N), 1), 0)
                )
                buf[jslot, r, pl.ds(col, 1), :] = g_ref[pl.ds(base + r, 1), :]
            _out_copy(j, jslot, out_ref, buf, osem).start()

        return carry

    lax.fori_loop(0, NBLK + 1, step, 0)
    for t in range(NBUF):
        j = NBLK - NBUF + t
        _out_copy(j, j % NBUF, out_ref, buf, osem).wait()


def _merge(tokenized_text, embedded_text, gathered):
    return pl.pallas_call(
        _pipe_body,
        in_specs=[
            pl.BlockSpec(memory_space=pltpu.VMEM),
            pl.BlockSpec(memory_space=pl.ANY),
            pl.BlockSpec(memory_space=pltpu.VMEM),
        ],
        out_specs=pl.BlockSpec(memory_space=pl.ANY),
        out_shape=jax.ShapeDtypeStruct((B, N, D), jnp.float32),
        scratch_shapes=[
            pltpu.VMEM((NBUF, BB, N, D), jnp.float32),
            pltpu.SemaphoreType.DMA((NBUF,)),
            pltpu.SemaphoreType.DMA((NBUF,)),
        ],
    )(tokenized_text, embedded_text, gathered)


def kernel(tokenized_text, embedded_text, name, params):
    params2d = params.reshape(NUM_NAMES, D)
    gathered = _sc_gather()(name, params2d)
    return _merge(tokenized_text, embedded_text, gathered)
